# Initial kernel scaffold; baseline (speedup 1.0000x reference)
import jax
import jax.numpy as jnp
from jax.experimental import pallas as pl


def kernel(input_ids, attention_mask, labels, edge_index, enc_emb, node_features, W1, b1, W2, b2):
    raise NotImplementedError("write your pallas kernel here")


# speedup vs baseline: 6.1930x; 6.1930x over previous
import functools

import jax
import jax.numpy as jnp
from jax import lax
from jax.experimental import pallas as pl
from jax.experimental.pallas import tpu as pltpu
from jax.experimental.pallas import tpu_sc as plsc

N_NODES = 10000
HID = 128
N_CLASSES = 2000
NPROTO = 5
MARGIN_V = -0.1
LOSS_WV = 0.1
BATCH = 1024

NC, NS, LANES = 2, 16, 16
NW = NC * NS
NPAD = 10240
ROWS_PT = NPAD // NS
E_REAL = 320000
EPT = 10240
EPAD = NW * EPT
CH = 128
NCHUNK = EPT // CH
IDS_PT = BATCH // NW

def _sc_deg_emb_body(dst_hbm, ones_hbm, ids_hbm, emb_hbm, deg_out, hid_out,
                     idx_v, ones_v, zbuf, ids_v, erows_v, acc_sh, sem):
    c = lax.axis_index("c")
    s = lax.axis_index("s")
    wid = c * NS + s

    pltpu.sync_copy(ids_hbm.at[pl.ds(wid * IDS_PT, IDS_PT)], ids_v)
    pltpu.async_copy(emb_hbm.at[ids_v], erows_v, sem).wait()
    pltpu.sync_copy(erows_v, hid_out.at[pl.ds(wid * IDS_PT, IDS_PT)])

    pltpu.sync_copy(ones_hbm, ones_v)

    def _zrow(i, carry):
        for j in range(HID // LANES):
            zbuf[i, pl.ds(j * LANES, LANES)] = jnp.zeros((LANES,), jnp.float32)
        return carry
    lax.fori_loop(0, CH, _zrow, 0)
    rbase = s * ROWS_PT

    def _zcp(k, carry):
        pltpu.sync_copy(zbuf, acc_sh.at[pl.ds(rbase + k * CH, CH)])
        return carry
    lax.fori_loop(0, ROWS_PT // CH, _zcp, 0)
    plsc.subcore_barrier()

    ebase = wid * EPT

    def _chunk(i, carry):
        pltpu.sync_copy(dst_hbm.at[pl.ds(ebase + i * CH, CH)], idx_v)
        pltpu.sync_copy(ones_v, acc_sh.at[idx_v], add=True)
        return carry
    lax.fori_loop(0, NCHUNK, _chunk, 0)
    plsc.subcore_barrier()

    pltpu.sync_copy(acc_sh.at[pl.ds(rbase, ROWS_PT)],
                    deg_out.at[pl.ds(c * NPAD + rbase, ROWS_PT)])


def _sc_edge_scatter_body(g_hbm, src_hbm, dst_hbm, acc_out,
                          sidx, didx, rows_v, zbuf, acc_sh, sem):
    c = lax.axis_index("c")
    s = lax.axis_index("s")
    wid = c * NS + s

    def _zrow(i, carry):
        for j in range(HID // LANES):
            zbuf[i, pl.ds(j * LANES, LANES)] = jnp.zeros((LANES,), jnp.float32)
        return carry
    lax.fori_loop(0, CH, _zrow, 0)
    rbase = s * ROWS_PT

    def _zcp(k, carry):
        pltpu.sync_copy(zbuf, acc_sh.at[pl.ds(rbase + k * CH, CH)])
        return carry
    lax.fori_loop(0, ROWS_PT // CH, _zcp, 0)
    plsc.subcore_barrier()

    ebase = wid * EPT

    def _chunk(i, carry):
        pltpu.sync_copy(src_hbm.at[pl.ds(ebase + i * CH, CH)], sidx)
        pltpu.sync_copy(dst_hbm.at[pl.ds(ebase + i * CH, CH)], didx)
        pltpu.async_copy(g_hbm.at[sidx], rows_v, sem).wait()
        pltpu.sync_copy(rows_v, acc_sh.at[didx], add=True)
        return carry
    lax.fori_loop(0, NCHUNK, _chunk, 0)
    plsc.subcore_barrier()

    pltpu.sync_copy(acc_sh.at[pl.ds(rbase, ROWS_PT)],
                    acc_out.at[pl.ds(c * NPAD + rbase, ROWS_PT)])


@functools.cache
def _sc_kernels():
    mesh = plsc.VectorSubcoreMesh(
        core_axis_name="c", subcore_axis_name="s",
        num_cores=NC, num_subcores=NS)
    deg_emb = pl.kernel(
        _sc_deg_emb_body,
        out_type=(
            jax.ShapeDtypeStruct((NC * NPAD, HID), jnp.float32),
            jax.ShapeDtypeStruct((BATCH, HID), jnp.float32),
        ),
        mesh=mesh,
        scratch_types=[
            pltpu.VMEM((CH,), jnp.int32),
            pltpu.VMEM((CH, HID), jnp.float32),
            pltpu.VMEM((CH, HID), jnp.float32),
            pltpu.VMEM((IDS_PT,), jnp.int32),
            pltpu.VMEM((IDS_PT, HID), jnp.float32),
            pltpu.VMEM_SHARED((NPAD, HID), jnp.float32),
            pltpu.SemaphoreType.DMA,
        ],
    )
    edge_scatter = pl.kernel(
        _sc_edge_scatter_body,
        out_type=jax.ShapeDtypeStruct((NC * NPAD, HID), jnp.float32),
        mesh=mesh,
        scratch_types=[
            pltpu.VMEM((CH,), jnp.int32),
            pltpu.VMEM((CH,), jnp.int32),
            pltpu.VMEM((CH, HID), jnp.float32),
            pltpu.VMEM((CH, HID), jnp.float32),
            pltpu.VMEM_SHARED((NPAD, HID), jnp.float32),
            pltpu.SemaphoreType.DMA,
        ],
    )
    return deg_emb, edge_scatter


def _bdot(a, b):
    return jnp.dot(a, b, preferred_element_type=jnp.float32,
                   precision=jax.lax.Precision.HIGHEST)


def _tc_pre1(x_ref, w_ref, d0_ref, d1_ref, g_ref, dis_ref):
    deg = d0_ref[...] + d1_ref[...] + 1.0
    dis = 1.0 / jnp.sqrt(deg)
    dis_ref[...] = dis
    g_ref[...] = dis * _bdot(x_ref[...], w_ref[...])


def _tc_mid(a0_ref, a1_ref, g_ref, dis_ref, b_ref, w_ref, out_ref):
    x = dis_ref[...] * (a0_ref[...] + a1_ref[...] + g_ref[...]) + b_ref[...]
    x = jnp.maximum(x, 0.0)
    out_ref[...] = dis_ref[...] * _bdot(x, w_ref[...])


def _tc_post(a0_ref, a1_ref, g_ref, dis_ref, b_ref, pn_ref, ps_ref):
    x = dis_ref[...] * (a0_ref[...] + a1_ref[...] + g_ref[...]) + b_ref[...]
    x = jnp.maximum(x, 0.0)
    nrm = jnp.sqrt(jnp.sum(x * x, axis=1, keepdims=True))
    pn = x / jnp.maximum(nrm, 1e-12)
    pn_ref[...] = pn
    ps_ref[...] = jnp.sum(pn, axis=0, keepdims=True)


BB = 256


def _tc_sim_loss(hid_ref, pnt_ref, lab_ref, ps_ref, out_ref, loss_ref, acc_ref):
    b = pl.program_id(0)
    h = hid_ref[...]
    hn = h / jnp.maximum(jnp.sqrt(jnp.sum(h * h, axis=1, keepdims=True)), 1e-12)
    m = _bdot(hn, pnt_ref[0])
    for j in range(1, NPROTO):
        m = jnp.maximum(m, _bdot(hn, pnt_ref[j]))
    outv = (1.0 + m) * 0.5
    out_ref[...] = outv

    lab = lab_ref[...]
    col = lax.broadcasted_iota(jnp.int32, (BB, N_CLASSES), 1)
    true_s = jnp.sum(jnp.where(col == lab, outv, 0.0), axis=1, keepdims=True)
    pred_s = jnp.max(outv, axis=1, keepdims=True)
    esum = jnp.sum(jnp.exp(outv), axis=1, keepdims=True)
    msk = (true_s - pred_s > MARGIN_V).astype(jnp.float32)
    ce = (jnp.log(esum) - true_s) * msk
    ce_sum = jnp.sum(ce)
    m_sum = jnp.sum(msk)

    @pl.when(b == 0)
    def _():
        acc_ref[0] = ce_sum
        acc_ref[1] = m_sum

    @pl.when(b > 0)
    def _():
        acc_ref[0] += ce_sum
        acc_ref[1] += m_sum

    @pl.when(b == pl.num_programs(0) - 1)
    def _():
        psv = ps_ref[...]
        disp = -jnp.sum(psv * psv) / (float(N_NODES) * float(N_NODES))
        nm = acc_ref[1]
        total = acc_ref[0] / jnp.maximum(nm, 1.0) + LOSS_WV * disp
        loss_ref[...] = jnp.broadcast_to(jnp.where(nm == 0.0, 0.0, total),
                                         (1, 1))


_pre1 = pl.pallas_call(
    _tc_pre1,
    out_shape=(jax.ShapeDtypeStruct((N_NODES, HID), jnp.float32),
               jax.ShapeDtypeStruct((N_NODES, 1), jnp.float32)),
)

_mid = pl.pallas_call(
    _tc_mid,
    out_shape=jax.ShapeDtypeStruct((N_NODES, HID), jnp.float32),
)

_post = pl.pallas_call(
    _tc_post,
    out_shape=(jax.ShapeDtypeStruct((N_NODES, HID), jnp.float32),
               jax.ShapeDtypeStruct((1, HID), jnp.float32)),
)

_sim_loss = pl.pallas_call(
    _tc_sim_loss,
    grid=(BATCH // BB,),
    in_specs=[
        pl.BlockSpec((BB, HID), lambda b: (b, 0)),
        pl.BlockSpec((NPROTO, HID, N_CLASSES), lambda b: (0, 0, 0)),
        pl.BlockSpec((BB, 1), lambda b: (b, 0)),
        pl.BlockSpec((1, HID), lambda b: (0, 0)),
    ],
    out_specs=[
        pl.BlockSpec((BB, N_CLASSES), lambda b: (b, 0)),
        pl.BlockSpec((1, 1), lambda b: (0, 0)),
    ],
    out_shape=(jax.ShapeDtypeStruct((BATCH, N_CLASSES), jnp.float32),
               jax.ShapeDtypeStruct((1, 1), jnp.float32)),
    scratch_shapes=[pltpu.SMEM((2,), jnp.float32)],
)


def kernel(input_ids, attention_mask, labels, edge_index, enc_emb,
           node_features, W1, b1, W2, b2):
    del attention_mask
    ids = input_ids[:, 0].astype(jnp.int32)
    npad = EPAD - E_REAL
    srcp = jnp.concatenate(
        [edge_index[0].astype(jnp.int32), jnp.zeros((npad,), jnp.int32)])
    dstp = jnp.concatenate(
        [edge_index[1].astype(jnp.int32),
         jnp.full((npad,), N_NODES, jnp.int32)])

    sc_deg_emb, sc_edge_scatter = _sc_kernels()
    ones_rows = jnp.ones((CH, HID), jnp.float32)
    deg_part, hidden = sc_deg_emb(dstp, ones_rows, ids, enc_emb)
    d0 = deg_part[:N_NODES, :1]
    d1 = deg_part[NPAD:NPAD + N_NODES, :1]

    g1, dis = _pre1(node_features, W1, d0, d1)
    acc1 = sc_edge_scatter(g1, srcp, dstp)
    g2 = _mid(acc1[:N_NODES], acc1[NPAD:NPAD + N_NODES], g1, dis,
              b1.reshape(1, HID), W2)
    acc2 = sc_edge_scatter(g2, srcp, dstp)
    pn, ps = _post(acc2[:N_NODES], acc2[NPAD:NPAD + N_NODES], g2, dis,
                   b2.reshape(1, HID))

    pnt = pn.reshape(N_CLASSES, NPROTO, HID).transpose(1, 2, 0)
    out, loss = _sim_loss(hidden, pnt, labels.reshape(BATCH, 1).astype(jnp.int32), ps)
    return out, loss[0, 0]


# speedup vs baseline: 8.3441x; 1.3474x over previous
import functools

import jax
import jax.numpy as jnp
from jax import lax
from jax.experimental import pallas as pl
from jax.experimental.pallas import tpu as pltpu
from jax.experimental.pallas import tpu_sc as plsc

N_NODES = 10000
HID = 128
N_CLASSES = 2000
NPROTO = 5
MARGIN_V = -0.1
LOSS_WV = 0.1
BATCH = 1024

NC, NS, LANES = 2, 16, 16
NW = NC * NS
NPAD = 10240
ROWS_PT = NPAD // NS
E_REAL = 320000
EPT = 10240
EPAD = NW * EPT
CH = 128
NCHUNK = EPT // CH
IBLK = 16
IDS_PT = BATCH // NW

def _sc_deg_emb_body(dst_hbm, ones_hbm, ids_hbm, emb_hbm, deg_out, hid_out,
                     idx_v, ones_v, ids_v, erows_v, acc_sh, sem):
    c = lax.axis_index("c")
    s = lax.axis_index("s")
    wid = c * NS + s

    pltpu.sync_copy(ids_hbm.at[pl.ds(wid * IDS_PT, IDS_PT)], ids_v)
    pltpu.async_copy(emb_hbm.at[ids_v], erows_v, sem).wait()
    pltpu.sync_copy(erows_v, hid_out.at[pl.ds(wid * IDS_PT, IDS_PT)])

    pltpu.sync_copy(dst_hbm.at[pl.ds(wid * NCHUNK, NCHUNK)], idx_v)

    def _zrow(i, carry):
        for j in range(HID // LANES):
            ones_v[i, pl.ds(j * LANES, LANES)] = jnp.zeros((LANES,),
                                                           jnp.float32)
        return carry
    lax.fori_loop(0, CH, _zrow, 0)
    rbase = s * ROWS_PT

    def _zcp(k, carry):
        pltpu.sync_copy(ones_v, acc_sh.at[pl.ds(rbase + k * CH, CH)])
        return carry
    lax.fori_loop(0, ROWS_PT // CH, _zcp, 0)
    pltpu.sync_copy(ones_hbm, ones_v)
    plsc.subcore_barrier()

    def _chunk(i, carry):
        pltpu.sync_copy(ones_v, acc_sh.at[idx_v.at[i]], add=True)
        return carry
    lax.fori_loop(0, NCHUNK, _chunk, 0)
    plsc.subcore_barrier()

    pltpu.sync_copy(acc_sh.at[pl.ds(rbase, ROWS_PT)],
                    deg_out.at[pl.ds(c * NPAD + rbase, ROWS_PT)])


def _sc_edge_scatter_body(g_hbm, src_hbm, dst_hbm, acc_out,
                          sidx, didx, rows0, rows1, acc_sh, sem0, sem1):
    c = lax.axis_index("c")
    s = lax.axis_index("s")
    wid = c * NS + s

    def _zrow(i, carry):
        for j in range(HID // LANES):
            rows0[i, pl.ds(j * LANES, LANES)] = jnp.zeros((LANES,),
                                                          jnp.float32)
        return carry
    lax.fori_loop(0, CH, _zrow, 0)
    rbase = s * ROWS_PT

    def _zcp(k, carry):
        pltpu.sync_copy(rows0, acc_sh.at[pl.ds(rbase + k * CH, CH)])
        return carry
    lax.fori_loop(0, ROWS_PT // CH, _zcp, 0)
    plsc.subcore_barrier()

    def _blk(b, carry):
        base = wid * NCHUNK + b * IBLK
        pltpu.sync_copy(src_hbm.at[pl.ds(base, IBLK)], sidx)
        pltpu.sync_copy(dst_hbm.at[pl.ds(base, IBLK)], didx)
        pltpu.async_copy(g_hbm.at[sidx.at[0]], rows0, sem0)

        def _pair(k, carry2):
            i0 = 2 * k
            pltpu.async_copy(g_hbm.at[sidx.at[i0 + 1]], rows1, sem1)
            pltpu.make_async_copy(g_hbm.at[sidx.at[i0]], rows0, sem0).wait()
            pltpu.sync_copy(rows0, acc_sh.at[didx.at[i0]], add=True)

            @pl.when(k < IBLK // 2 - 1)
            def _():
                pltpu.async_copy(g_hbm.at[sidx.at[i0 + 2]], rows0, sem0)

            pltpu.make_async_copy(g_hbm.at[sidx.at[i0 + 1]], rows1,
                                  sem1).wait()
            pltpu.sync_copy(rows1, acc_sh.at[didx.at[i0 + 1]], add=True)
            return carry2
        lax.fori_loop(0, IBLK // 2, _pair, 0)
        return carry
    lax.fori_loop(0, NCHUNK // IBLK, _blk, 0)
    plsc.subcore_barrier()

    pltpu.sync_copy(acc_sh.at[pl.ds(rbase, ROWS_PT)],
                    acc_out.at[pl.ds(c * NPAD + rbase, ROWS_PT)])


@functools.cache
def _sc_kernels():
    mesh = plsc.VectorSubcoreMesh(
        core_axis_name="c", subcore_axis_name="s",
        num_cores=NC, num_subcores=NS)
    deg_emb = pl.kernel(
        _sc_deg_emb_body,
        out_type=(
            jax.ShapeDtypeStruct((NC * NPAD, HID), jnp.float32),
            jax.ShapeDtypeStruct((BATCH, HID), jnp.float32),
        ),
        mesh=mesh,
        scratch_types=[
            pltpu.VMEM((NCHUNK, CH), jnp.int32),
            pltpu.VMEM((CH, HID), jnp.float32),
            pltpu.VMEM((IDS_PT,), jnp.int32),
            pltpu.VMEM((IDS_PT, HID), jnp.float32),
            pltpu.VMEM_SHARED((NPAD, HID), jnp.float32),
            pltpu.SemaphoreType.DMA,
        ],
    )
    edge_scatter = pl.kernel(
        _sc_edge_scatter_body,
        out_type=jax.ShapeDtypeStruct((NC * NPAD, HID), jnp.float32),
        mesh=mesh,
        scratch_types=[
            pltpu.VMEM((IBLK, CH), jnp.int32),
            pltpu.VMEM((IBLK, CH), jnp.int32),
            pltpu.VMEM((CH, HID), jnp.float32),
            pltpu.VMEM((CH, HID), jnp.float32),
            pltpu.VMEM_SHARED((NPAD, HID), jnp.float32),
            pltpu.SemaphoreType.DMA,
            pltpu.SemaphoreType.DMA,
        ],
    )
    return deg_emb, edge_scatter


def _bdot(a, b):
    return jnp.dot(a, b, preferred_element_type=jnp.float32,
                   precision=jax.lax.Precision.HIGHEST)


def _tc_pre1(x_ref, w_ref, d0_ref, d1_ref, g_ref, dis_ref):
    deg = d0_ref[...] + d1_ref[...] + 1.0
    dis = 1.0 / jnp.sqrt(deg)
    dis_ref[...] = dis
    g_ref[...] = dis * _bdot(x_ref[...], w_ref[...])


def _tc_mid(a0_ref, a1_ref, g_ref, dis_ref, b_ref, w_ref, out_ref):
    x = dis_ref[...] * (a0_ref[...] + a1_ref[...] + g_ref[...]) + b_ref[...]
    x = jnp.maximum(x, 0.0)
    out_ref[...] = dis_ref[...] * _bdot(x, w_ref[...])


def _tc_post(a0_ref, a1_ref, g_ref, dis_ref, b_ref, pn_ref, ps_ref):
    x = dis_ref[...] * (a0_ref[...] + a1_ref[...] + g_ref[...]) + b_ref[...]
    x = jnp.maximum(x, 0.0)
    nrm = jnp.sqrt(jnp.sum(x * x, axis=1, keepdims=True))
    pn = x / jnp.maximum(nrm, 1e-12)
    pn_ref[...] = pn
    ps_ref[...] = jnp.sum(pn, axis=0, keepdims=True)


BB = 256


def _tc_sim_loss(hid_ref, pnt_ref, lab_ref, ps_ref, out_ref, loss_ref, acc_ref):
    b = pl.program_id(0)
    h = hid_ref[...]
    hn = h / jnp.maximum(jnp.sqrt(jnp.sum(h * h, axis=1, keepdims=True)), 1e-12)
    m = _bdot(hn, pnt_ref[0])
    for j in range(1, NPROTO):
        m = jnp.maximum(m, _bdot(hn, pnt_ref[j]))
    outv = (1.0 + m) * 0.5
    out_ref[...] = outv

    lab = lab_ref[...]
    col = lax.broadcasted_iota(jnp.int32, (BB, N_CLASSES), 1)
    true_s = jnp.sum(jnp.where(col == lab, outv, 0.0), axis=1, keepdims=True)
    pred_s = jnp.max(outv, axis=1, keepdims=True)
    esum = jnp.sum(jnp.exp(outv), axis=1, keepdims=True)
    msk = (true_s - pred_s > MARGIN_V).astype(jnp.float32)
    ce = (jnp.log(esum) - true_s) * msk
    ce_sum = jnp.sum(ce)
    m_sum = jnp.sum(msk)

    @pl.when(b == 0)
    def _():
        acc_ref[0] = ce_sum
        acc_ref[1] = m_sum

    @pl.when(b > 0)
    def _():
        acc_ref[0] += ce_sum
        acc_ref[1] += m_sum

    @pl.when(b == pl.num_programs(0) - 1)
    def _():
        psv = ps_ref[...]
        disp = -jnp.sum(psv * psv) / (float(N_NODES) * float(N_NODES))
        nm = acc_ref[1]
        total = acc_ref[0] / jnp.maximum(nm, 1.0) + LOSS_WV * disp
        loss_ref[...] = jnp.broadcast_to(jnp.where(nm == 0.0, 0.0, total),
                                         (1, 1))


_pre1 = pl.pallas_call(
    _tc_pre1,
    out_shape=(jax.ShapeDtypeStruct((N_NODES, HID), jnp.float32),
               jax.ShapeDtypeStruct((N_NODES, 1), jnp.float32)),
)

_mid = pl.pallas_call(
    _tc_mid,
    out_shape=jax.ShapeDtypeStruct((N_NODES, HID), jnp.float32),
)

_post = pl.pallas_call(
    _tc_post,
    out_shape=(jax.ShapeDtypeStruct((N_NODES, HID), jnp.float32),
               jax.ShapeDtypeStruct((1, HID), jnp.float32)),
)

_sim_loss = pl.pallas_call(
    _tc_sim_loss,
    grid=(BATCH // BB,),
    in_specs=[
        pl.BlockSpec((BB, HID), lambda b: (b, 0)),
        pl.BlockSpec((NPROTO, HID, N_CLASSES), lambda b: (0, 0, 0)),
        pl.BlockSpec((BB, 1), lambda b: (b, 0)),
        pl.BlockSpec((1, HID), lambda b: (0, 0)),
    ],
    out_specs=[
        pl.BlockSpec((BB, N_CLASSES), lambda b: (b, 0)),
        pl.BlockSpec((1, 1), lambda b: (0, 0)),
    ],
    out_shape=(jax.ShapeDtypeStruct((BATCH, N_CLASSES), jnp.float32),
               jax.ShapeDtypeStruct((1, 1), jnp.float32)),
    scratch_shapes=[pltpu.SMEM((2,), jnp.float32)],
)


def kernel(input_ids, attention_mask, labels, edge_index, enc_emb,
           node_features, W1, b1, W2, b2):
    del attention_mask
    ids = input_ids[:, 0].astype(jnp.int32)
    npad = EPAD - E_REAL
    srcp = jnp.concatenate(
        [edge_index[0].astype(jnp.int32),
         jnp.zeros((npad,), jnp.int32)]).reshape(NW * NCHUNK, CH)
    dstp = jnp.concatenate(
        [edge_index[1].astype(jnp.int32),
         jnp.full((npad,), N_NODES, jnp.int32)]).reshape(NW * NCHUNK, CH)

    sc_deg_emb, sc_edge_scatter = _sc_kernels()
    ones_rows = jnp.ones((CH, HID), jnp.float32)
    deg_part, hidden = sc_deg_emb(dstp, ones_rows, ids, enc_emb)
    d0 = deg_part[:N_NODES, :1]
    d1 = deg_part[NPAD:NPAD + N_NODES, :1]

    g1, dis = _pre1(node_features, W1, d0, d1)
    acc1 = sc_edge_scatter(g1, srcp, dstp)
    g2 = _mid(acc1[:N_NODES], acc1[NPAD:NPAD + N_NODES], g1, dis,
              b1.reshape(1, HID), W2)
    acc2 = sc_edge_scatter(g2, srcp, dstp)
    pn, ps = _post(acc2[:N_NODES], acc2[NPAD:NPAD + N_NODES], g2, dis,
                   b2.reshape(1, HID))

    pnt = pn.reshape(N_CLASSES, NPROTO, HID).transpose(1, 2, 0)
    out, loss = _sim_loss(hidden, pnt, labels.reshape(BATCH, 1).astype(jnp.int32), ps)
    return out, loss[0, 0]
